# baseline (speedup 1.0000x reference)
import jax
import jax.numpy as jnp
from jax.experimental import pallas as pl
from jax.experimental.pallas import tpu as pltpu

F32 = jnp.float32
BF16 = jnp.bfloat16

B = 16
C_IN = 64
T = 4096
H = 512
D = 64
K = 1024
Q = 8
BETA = 0.25
L = 1022
LP = 1024
NVALID = B * L * D


def _dot(a, b, dims=(((1,), (0,)), ((), ()))):
    return jax.lax.dot_general(a, b, dims, preferred_element_type=F32)


def _fused_kernel(xt_ref, w1_ref, b1_ref, w2_ref, b2_ref, w3_ref, b3_ref,
                  w4_ref, b4_ref, cbb_ref, cbcat_ref, csq_ref,
                  wd1_ref, bd1_ref, w20_ref, w21_ref, w22_ref, w23_ref,
                  db2_ref, v0_ref, v1_ref, v2_ref, v3_ref, db3_ref,
                  wd4_ref, db4_ref,
                  o0_ref, o1_ref, o2_ref, o3_ref, loss_ref):
    bidx = pl.program_id(0)

    xt = xt_ref[0].astype(BF16)
    y1 = jnp.maximum(_dot(xt, w1_ref[...]) + b1_ref[...], 0.0)
    z2 = y1.astype(BF16).reshape(2048, 256)
    z2n = pltpu.roll(z2, 2047, axis=0)
    w2 = w2_ref[...]
    y2 = _dot(jnp.concatenate([z2, z2n], axis=1), w2)
    y2 = jnp.maximum(y2 + b2_ref[...], 0.0)
    p = y2.astype(BF16).reshape(1024, 512)
    pn = pltpu.roll(p, 1023, axis=0)
    w3 = w3_ref[...]
    y3 = _dot(jnp.concatenate([p, pn], axis=1), w3)
    y3 = jnp.maximum(y3 + b3_ref[...], 0.0)
    z = _dot(y3.astype(BF16), w4_ref[...]) + b4_ref[...]

    rowmask = jax.lax.broadcasted_iota(jnp.int32, (LP, 1), 0) < L
    iota_k = jax.lax.broadcasted_iota(jnp.int32, (LP, K), 1)
    r = z
    acc = jnp.zeros_like(z)
    loss = jnp.float32(0.0)
    for q in range(Q):
        cbb = cbb_ref[q]
        csq = csq_ref[q]
        rowsq = jnp.sum(r * r, axis=1, keepdims=True)
        prod = _dot(r.astype(BF16), cbb, (((1,), (1,)), ((), ())))
        d2 = rowsq - 2.0 * prod + csq
        idx = jnp.argmin(d2, axis=1).astype(jnp.int32)[:, None]
        onehot = (iota_k == idx).astype(BF16)
        parts = _dot(onehot, cbcat_ref[q])
        qv = (parts[:, :D] + parts[:, D:2 * D]) + parts[:, 2 * D:]
        diff = qv - r
        loss = loss + jnp.sum(jnp.where(rowmask, diff * diff, 0.0))
        qst = r + diff
        acc = acc + qst
        r = r - qst
    zq = acc * rowmask.astype(F32)

    @pl.when(bidx == 0)
    def _():
        loss_ref[0, 0] = jnp.float32(0.0)

    loss_ref[0, 0] += loss * jnp.float32((1.0 + BETA) / NVALID)

    iota = jax.lax.broadcasted_iota(jnp.int32, (LP, 1), 0)
    h1 = jnp.maximum(_dot(zq.astype(BF16), wd1_ref[...]) + bd1_ref[...], 0.0)
    h1 = jnp.where(iota < L, h1, 0.0).astype(BF16)
    h1p = jnp.where(iota == 0, 0, pltpu.roll(h1, 1, axis=0))
    e2 = jnp.maximum(_dot(h1, w20_ref[...]) + _dot(h1p, w22_ref[...])
                     + db2_ref[...], 0.0)
    o2 = jnp.maximum(_dot(h1, w21_ref[...]) + _dot(h1p, w23_ref[...])
                     + db2_ref[...], 0.0)
    e2 = jnp.where(iota < L + 1, e2, 0.0).astype(BF16)
    o2 = jnp.where(iota < L + 1, o2, 0.0).astype(BF16)
    e2p = jnp.where(iota == 0, 0, pltpu.roll(e2, 1, axis=0))
    o2p = jnp.where(iota == 0, 0, pltpu.roll(o2, 1, axis=0))
    b3 = db3_ref[...]
    sa = jnp.maximum(_dot(e2, v0_ref[...]) + _dot(o2p, v2_ref[...]) + b3, 0.0)
    sb = jnp.maximum(_dot(e2, v1_ref[...]) + _dot(o2p, v3_ref[...]) + b3, 0.0)
    sc = jnp.maximum(_dot(o2, v0_ref[...]) + _dot(e2, v2_ref[...]) + b3, 0.0)
    sd = jnp.maximum(_dot(o2, v1_ref[...]) + _dot(e2, v3_ref[...]) + b3, 0.0)
    wd4 = wd4_ref[...]
    b4 = db4_ref[...]
    o0_ref[0] = jnp.transpose(_dot(sa.astype(BF16), wd4) + b4)
    o1_ref[0] = jnp.transpose(_dot(sb.astype(BF16), wd4) + b4)
    o2_ref[0] = jnp.transpose(_dot(sc.astype(BF16), wd4) + b4)
    o3_ref[0] = jnp.transpose(_dot(sd.astype(BF16), wd4) + b4)


def _full_spec(shape):
    return pl.BlockSpec(shape, lambda b: (0,) * len(shape))


def _split3(cb):
    hi = cb.astype(BF16)
    lo32 = cb - hi.astype(F32)
    lo = lo32.astype(BF16)
    llo = (lo32 - lo.astype(F32)).astype(BF16)
    return jnp.concatenate([hi, lo, llo], axis=-1)


def kernel(x, codebooks, e_w1, e_b1, e_w2, e_b2, e_w3, e_b3, e_w4, e_b4,
           d_w1, d_b1, d_w2, d_b2, d_w3, d_b3, d_w4, d_b4):
    w1p = e_w1[:, :, 0].T.astype(BF16)
    w2p = jnp.transpose(e_w2, (2, 1, 0)).reshape(512, 256).astype(BF16)
    w3p = jnp.transpose(e_w3, (2, 1, 0)).reshape(1024, 512).astype(BF16)
    w4p = e_w4[:, :, 0].T.astype(BF16)
    xt = jnp.transpose(x, (0, 2, 1))
    wd1 = d_w1[:, :, 0].astype(BF16)
    w20, w21, w22, w23 = (d_w2[:, :, k].astype(BF16) for k in range(4))
    v0, v1, v2, v3 = (d_w3[:, :, k].astype(BF16) for k in range(4))
    wd4 = d_w4[:, :, 0].astype(BF16)

    outs = pl.pallas_call(
        _fused_kernel,
        grid=(B,),
        in_specs=[
            pl.BlockSpec((1, T, C_IN), lambda b: (b, 0, 0)),
            _full_spec((C_IN, 128)), _full_spec((1, 128)),
            _full_spec((512, 256)), _full_spec((1, 256)),
            _full_spec((1024, 512)), _full_spec((1, 512)),
            _full_spec((512, D)), _full_spec((1, D)),
            _full_spec((Q, K, D)),
            _full_spec((Q, K, 3 * D)),
            _full_spec((Q, 1, K)),
            _full_spec((D, H)), _full_spec((1, H)),
            _full_spec((H, 256)), _full_spec((H, 256)),
            _full_spec((H, 256)), _full_spec((H, 256)), _full_spec((1, 256)),
            _full_spec((256, 128)), _full_spec((256, 128)),
            _full_spec((256, 128)), _full_spec((256, 128)), _full_spec((1, 128)),
            _full_spec((128, C_IN)), _full_spec((1, C_IN)),
        ],
        out_specs=[pl.BlockSpec((1, C_IN, LP), lambda b: (b, 0, 0))] * 4
        + [pl.BlockSpec(memory_space=pltpu.SMEM)],
        out_shape=[jax.ShapeDtypeStruct((B, C_IN, LP), F32)] * 4
        + [jax.ShapeDtypeStruct((1, 1), F32)],
    )(xt, w1p, e_b1[None, :], w2p, e_b2[None, :], w3p, e_b3[None, :],
      w4p, e_b4[None, :], codebooks.astype(BF16), _split3(codebooks),
      jnp.sum(codebooks * codebooks, axis=-1)[:, None, :],
      wd1, d_b1[None, :], w20, w21, w22, w23, d_b2[None, :],
      v0, v1, v2, v3, d_b3[None, :], wd4, d_b4[None, :])

    o0, o1, o2, o3, loss = outs
    xh = jnp.stack([o0, o1, o2, o3], axis=3)
    x_hat = xh.reshape(B, C_IN, 4 * LP)[:, :, :2 * (2 * L + 2) + 2]
    return x_hat, loss[0, 0]


# speedup vs baseline: 1.2496x; 1.2496x over previous
import jax
import jax.numpy as jnp
from jax.experimental import pallas as pl
from jax.experimental.pallas import tpu as pltpu

F32 = jnp.float32
BF16 = jnp.bfloat16

B = 16
C_IN = 64
T = 4096
H = 512
D = 64
K = 1024
Q = 8
BETA = 0.25
L = 1022
LP = 1024
NVALID = B * L * D


def _dot(a, b, dims=(((1,), (0,)), ((), ()))):
    return jax.lax.dot_general(a, b, dims, preferred_element_type=F32)


def _fused_kernel(xt_ref, w1_ref, b1_ref, w2_ref, b2_ref, w3_ref, b3_ref,
                  w4_ref, b4_ref, cbb_ref, cbcat_ref, csq_ref,
                  wd1_ref, bd1_ref, w20_ref, w21_ref, w22_ref, w23_ref,
                  db2_ref, v0_ref, v1_ref, v2_ref, v3_ref, db3_ref,
                  wd4_ref, db4_ref,
                  o0_ref, o1_ref, o2_ref, o3_ref, loss_ref):
    bidx = pl.program_id(0)

    xt = xt_ref[0].astype(BF16)
    y1 = jnp.maximum(_dot(xt, w1_ref[...]) + b1_ref[...], 0.0)
    z2 = y1.astype(BF16).reshape(2048, 256)
    z2n = pltpu.roll(z2, 2047, axis=0)
    w2 = w2_ref[...]
    y2 = _dot(jnp.concatenate([z2, z2n], axis=1), w2)
    y2 = jnp.maximum(y2 + b2_ref[...], 0.0)
    p = y2.astype(BF16).reshape(1024, 512)
    pn = pltpu.roll(p, 1023, axis=0)
    w3 = w3_ref[...]
    y3 = _dot(jnp.concatenate([p, pn], axis=1), w3)
    y3 = jnp.maximum(y3 + b3_ref[...], 0.0)
    z = _dot(y3.astype(BF16), w4_ref[...]) + b4_ref[...]

    rowmask = jax.lax.broadcasted_iota(jnp.int32, (LP, 1), 0) < L
    iota_k = jax.lax.broadcasted_iota(jnp.int32, (LP, K), 1)
    r = z
    acc = jnp.zeros_like(z)
    loss = jnp.float32(0.0)
    for q in range(Q):
        cbb = cbb_ref[q]
        csq = csq_ref[q]
        rowsq = jnp.sum(r * r, axis=1, keepdims=True)
        prod = _dot(r.astype(BF16), cbb, (((1,), (1,)), ((), ())))
        d2 = rowsq - 2.0 * prod + csq
        m = jnp.min(d2, axis=1, keepdims=True)
        idx = jnp.min(jnp.where(d2 == m, iota_k, K), axis=1, keepdims=True)
        onehot = (iota_k == idx).astype(BF16)
        parts = _dot(onehot, cbcat_ref[q])
        qv = (parts[:, :D] + parts[:, D:2 * D]) + parts[:, 2 * D:]
        diff = qv - r
        loss = loss + jnp.sum(jnp.where(rowmask, diff * diff, 0.0))
        qst = r + diff
        acc = acc + qst
        r = r - qst
    zq = acc * rowmask.astype(F32)

    @pl.when(bidx == 0)
    def _():
        loss_ref[0, 0] = jnp.float32(0.0)

    loss_ref[0, 0] += loss * jnp.float32((1.0 + BETA) / NVALID)

    iota = jax.lax.broadcasted_iota(jnp.int32, (LP, 1), 0)
    h1 = jnp.maximum(_dot(zq.astype(BF16), wd1_ref[...]) + bd1_ref[...], 0.0)
    h1 = jnp.where(iota < L, h1, 0.0).astype(BF16)
    h1p = jnp.where(iota == 0, 0, pltpu.roll(h1, 1, axis=0))
    e2 = jnp.maximum(_dot(h1, w20_ref[...]) + _dot(h1p, w22_ref[...])
                     + db2_ref[...], 0.0)
    o2 = jnp.maximum(_dot(h1, w21_ref[...]) + _dot(h1p, w23_ref[...])
                     + db2_ref[...], 0.0)
    e2 = jnp.where(iota < L + 1, e2, 0.0).astype(BF16)
    o2 = jnp.where(iota < L + 1, o2, 0.0).astype(BF16)
    e2p = jnp.where(iota == 0, 0, pltpu.roll(e2, 1, axis=0))
    o2p = jnp.where(iota == 0, 0, pltpu.roll(o2, 1, axis=0))
    b3 = db3_ref[...]
    sa = jnp.maximum(_dot(e2, v0_ref[...]) + _dot(o2p, v2_ref[...]) + b3, 0.0)
    sb = jnp.maximum(_dot(e2, v1_ref[...]) + _dot(o2p, v3_ref[...]) + b3, 0.0)
    sc = jnp.maximum(_dot(o2, v0_ref[...]) + _dot(e2, v2_ref[...]) + b3, 0.0)
    sd = jnp.maximum(_dot(o2, v1_ref[...]) + _dot(e2, v3_ref[...]) + b3, 0.0)
    wd4 = wd4_ref[...]
    b4 = db4_ref[...]
    o0_ref[0] = jnp.transpose(_dot(sa.astype(BF16), wd4) + b4)
    o1_ref[0] = jnp.transpose(_dot(sb.astype(BF16), wd4) + b4)
    o2_ref[0] = jnp.transpose(_dot(sc.astype(BF16), wd4) + b4)
    o3_ref[0] = jnp.transpose(_dot(sd.astype(BF16), wd4) + b4)


def _full_spec(shape):
    return pl.BlockSpec(shape, lambda b: (0,) * len(shape))


def _split3(cb):
    hi = cb.astype(BF16)
    lo32 = cb - hi.astype(F32)
    lo = lo32.astype(BF16)
    llo = (lo32 - lo.astype(F32)).astype(BF16)
    return jnp.concatenate([hi, lo, llo], axis=-1)


def kernel(x, codebooks, e_w1, e_b1, e_w2, e_b2, e_w3, e_b3, e_w4, e_b4,
           d_w1, d_b1, d_w2, d_b2, d_w3, d_b3, d_w4, d_b4):
    w1p = e_w1[:, :, 0].T.astype(BF16)
    w2p = jnp.transpose(e_w2, (2, 1, 0)).reshape(512, 256).astype(BF16)
    w3p = jnp.transpose(e_w3, (2, 1, 0)).reshape(1024, 512).astype(BF16)
    w4p = e_w4[:, :, 0].T.astype(BF16)
    xt = jnp.transpose(x, (0, 2, 1))
    wd1 = d_w1[:, :, 0].astype(BF16)
    w20, w21, w22, w23 = (d_w2[:, :, k].astype(BF16) for k in range(4))
    v0, v1, v2, v3 = (d_w3[:, :, k].astype(BF16) for k in range(4))
    wd4 = d_w4[:, :, 0].astype(BF16)

    outs = pl.pallas_call(
        _fused_kernel,
        grid=(B,),
        in_specs=[
            pl.BlockSpec((1, T, C_IN), lambda b: (b, 0, 0)),
            _full_spec((C_IN, 128)), _full_spec((1, 128)),
            _full_spec((512, 256)), _full_spec((1, 256)),
            _full_spec((1024, 512)), _full_spec((1, 512)),
            _full_spec((512, D)), _full_spec((1, D)),
            _full_spec((Q, K, D)),
            _full_spec((Q, K, 3 * D)),
            _full_spec((Q, 1, K)),
            _full_spec((D, H)), _full_spec((1, H)),
            _full_spec((H, 256)), _full_spec((H, 256)),
            _full_spec((H, 256)), _full_spec((H, 256)), _full_spec((1, 256)),
            _full_spec((256, 128)), _full_spec((256, 128)),
            _full_spec((256, 128)), _full_spec((256, 128)), _full_spec((1, 128)),
            _full_spec((128, C_IN)), _full_spec((1, C_IN)),
        ],
        out_specs=[pl.BlockSpec((1, C_IN, LP), lambda b: (b, 0, 0))] * 4
        + [pl.BlockSpec(memory_space=pltpu.SMEM)],
        out_shape=[jax.ShapeDtypeStruct((B, C_IN, LP), F32)] * 4
        + [jax.ShapeDtypeStruct((1, 1), F32)],
    )(xt, w1p, e_b1[None, :], w2p, e_b2[None, :], w3p, e_b3[None, :],
      w4p, e_b4[None, :], codebooks.astype(BF16), _split3(codebooks),
      jnp.sum(codebooks * codebooks, axis=-1)[:, None, :],
      wd1, d_b1[None, :], w20, w21, w22, w23, d_b2[None, :],
      v0, v1, v2, v3, d_b3[None, :], wd4, d_b4[None, :])

    o0, o1, o2, o3, loss = outs
    xh = jnp.stack([o0, o1, o2, o3], axis=3)
    x_hat = xh.reshape(B, C_IN, 4 * LP)[:, :, :2 * (2 * L + 2) + 2]
    return x_hat, loss[0, 0]


# speedup vs baseline: 1.2672x; 1.0141x over previous
import jax
import jax.numpy as jnp
from jax.experimental import pallas as pl
from jax.experimental.pallas import tpu as pltpu

F32 = jnp.float32
BF16 = jnp.bfloat16

B = 16
C_IN = 64
T = 4096
H = 512
D = 64
K = 1024
Q = 8
BETA = 0.25
L = 1022
LP = 1024
NVALID = B * L * D


def _dot(a, b, dims=(((1,), (0,)), ((), ()))):
    return jax.lax.dot_general(a, b, dims, preferred_element_type=F32)


def _fused_kernel(x_ref, w1_ref, b1_ref, w2_ref, b2_ref, w3_ref, b3_ref,
                  w4_ref, b4_ref, cb_ref,
                  wd1_ref, bd1_ref, w20_ref, w21_ref, w22_ref, w23_ref,
                  db2_ref, v0_ref, v1_ref, v2_ref, v3_ref, db3_ref,
                  wd4_ref, db4_ref,
                  o0_ref, o1_ref, o2_ref, o3_ref, loss_ref):
    bidx = pl.program_id(0)

    xt = jnp.transpose(x_ref[0].astype(BF16))
    y1 = jnp.maximum(_dot(xt, w1_ref[...]) + b1_ref[...], 0.0)
    z2 = y1.astype(BF16).reshape(2048, 256)
    z2n = pltpu.roll(z2, 2047, axis=0)
    w2 = w2_ref[...]
    y2 = _dot(jnp.concatenate([z2, z2n], axis=1), w2)
    y2 = jnp.maximum(y2 + b2_ref[...], 0.0)
    p = y2.astype(BF16).reshape(1024, 512)
    pn = pltpu.roll(p, 1023, axis=0)
    w3 = w3_ref[...]
    y3 = _dot(jnp.concatenate([p, pn], axis=1), w3)
    y3 = jnp.maximum(y3 + b3_ref[...], 0.0)
    z = _dot(y3.astype(BF16), w4_ref[...]) + b4_ref[...]

    cb_all = cb_ref[...]
    cbb_all = cb_all.astype(BF16)
    hi = cbb_all
    lo32 = cb_all - hi.astype(F32)
    lo = lo32.astype(BF16)
    llo = (lo32 - lo.astype(F32)).astype(BF16)
    cbcat_all = jnp.concatenate([hi, lo, llo], axis=-1)
    csq_all = jnp.sum(cb_all * cb_all, axis=-1)
    rowmask = jax.lax.broadcasted_iota(jnp.int32, (LP, 1), 0) < L
    iota_k = jax.lax.broadcasted_iota(jnp.int32, (LP, K), 1)
    r = z
    acc = jnp.zeros_like(z)
    loss = jnp.float32(0.0)
    for q in range(Q):
        cbb = cbb_all[q]
        csq = csq_all[q]
        rowsq = jnp.sum(r * r, axis=1, keepdims=True)
        prod = _dot(r.astype(BF16), cbb, (((1,), (1,)), ((), ())))
        d2 = rowsq - 2.0 * prod + csq[None, :]
        m = jnp.min(d2, axis=1, keepdims=True)
        idx = jnp.min(jnp.where(d2 == m, iota_k, K), axis=1, keepdims=True)
        onehot = (iota_k == idx).astype(BF16)
        parts = _dot(onehot, cbcat_all[q])
        qv = (parts[:, :D] + parts[:, D:2 * D]) + parts[:, 2 * D:]
        diff = qv - r
        loss = loss + jnp.sum(jnp.where(rowmask, diff * diff, 0.0))
        qst = r + diff
        acc = acc + qst
        r = r - qst
    zq = acc * rowmask.astype(F32)

    @pl.when(bidx == 0)
    def _():
        loss_ref[0, 0] = jnp.float32(0.0)

    loss_ref[0, 0] += loss * jnp.float32((1.0 + BETA) / NVALID)

    iota = jax.lax.broadcasted_iota(jnp.int32, (LP, 1), 0)
    h1 = jnp.maximum(_dot(zq.astype(BF16), wd1_ref[...]) + bd1_ref[...], 0.0)
    h1 = jnp.where(iota < L, h1, 0.0).astype(BF16)
    h1p = jnp.where(iota == 0, 0, pltpu.roll(h1, 1, axis=0))
    e2 = jnp.maximum(_dot(h1, w20_ref[...]) + _dot(h1p, w22_ref[...])
                     + db2_ref[...], 0.0)
    o2 = jnp.maximum(_dot(h1, w21_ref[...]) + _dot(h1p, w23_ref[...])
                     + db2_ref[...], 0.0)
    e2 = jnp.where(iota < L + 1, e2, 0.0).astype(BF16)
    o2 = jnp.where(iota < L + 1, o2, 0.0).astype(BF16)
    e2p = jnp.where(iota == 0, 0, pltpu.roll(e2, 1, axis=0))
    o2p = jnp.where(iota == 0, 0, pltpu.roll(o2, 1, axis=0))
    b3 = db3_ref[...]
    sa = jnp.maximum(_dot(e2, v0_ref[...]) + _dot(o2p, v2_ref[...]) + b3, 0.0)
    sb = jnp.maximum(_dot(e2, v1_ref[...]) + _dot(o2p, v3_ref[...]) + b3, 0.0)
    sc = jnp.maximum(_dot(o2, v0_ref[...]) + _dot(e2, v2_ref[...]) + b3, 0.0)
    sd = jnp.maximum(_dot(o2, v1_ref[...]) + _dot(e2, v3_ref[...]) + b3, 0.0)
    wd4 = wd4_ref[...]
    b4 = db4_ref[...]
    o0_ref[0] = jnp.transpose(_dot(sa.astype(BF16), wd4) + b4)
    o1_ref[0] = jnp.transpose(_dot(sb.astype(BF16), wd4) + b4)
    o2_ref[0] = jnp.transpose(_dot(sc.astype(BF16), wd4) + b4)
    o3_ref[0] = jnp.transpose(_dot(sd.astype(BF16), wd4) + b4)


def _full_spec(shape):
    return pl.BlockSpec(shape, lambda b: (0,) * len(shape))


def _split3(cb):
    hi = cb.astype(BF16)
    lo32 = cb - hi.astype(F32)
    lo = lo32.astype(BF16)
    llo = (lo32 - lo.astype(F32)).astype(BF16)
    return jnp.concatenate([hi, lo, llo], axis=-1)


def kernel(x, codebooks, e_w1, e_b1, e_w2, e_b2, e_w3, e_b3, e_w4, e_b4,
           d_w1, d_b1, d_w2, d_b2, d_w3, d_b3, d_w4, d_b4):
    w1p = e_w1[:, :, 0].T.astype(BF16)
    w2p = jnp.transpose(e_w2, (2, 1, 0)).reshape(512, 256).astype(BF16)
    w3p = jnp.transpose(e_w3, (2, 1, 0)).reshape(1024, 512).astype(BF16)
    w4p = e_w4[:, :, 0].T.astype(BF16)
    wd1 = d_w1[:, :, 0].astype(BF16)
    w20, w21, w22, w23 = (d_w2[:, :, k].astype(BF16) for k in range(4))
    v0, v1, v2, v3 = (d_w3[:, :, k].astype(BF16) for k in range(4))
    wd4 = d_w4[:, :, 0].astype(BF16)

    outs = pl.pallas_call(
        _fused_kernel,
        grid=(B,),
        in_specs=[
            pl.BlockSpec((1, C_IN, T), lambda b: (b, 0, 0)),
            _full_spec((C_IN, 128)), _full_spec((1, 128)),
            _full_spec((512, 256)), _full_spec((1, 256)),
            _full_spec((1024, 512)), _full_spec((1, 512)),
            _full_spec((512, D)), _full_spec((1, D)),
            _full_spec((Q, K, D)),
            _full_spec((D, H)), _full_spec((1, H)),
            _full_spec((H, 256)), _full_spec((H, 256)),
            _full_spec((H, 256)), _full_spec((H, 256)), _full_spec((1, 256)),
            _full_spec((256, 128)), _full_spec((256, 128)),
            _full_spec((256, 128)), _full_spec((256, 128)), _full_spec((1, 128)),
            _full_spec((128, C_IN)), _full_spec((1, C_IN)),
        ],
        out_specs=[pl.BlockSpec((1, C_IN, LP), lambda b: (b, 0, 0))] * 4
        + [pl.BlockSpec(memory_space=pltpu.SMEM)],
        out_shape=[jax.ShapeDtypeStruct((B, C_IN, LP), F32)] * 4
        + [jax.ShapeDtypeStruct((1, 1), F32)],
    )(x, w1p, e_b1[None, :], w2p, e_b2[None, :], w3p, e_b3[None, :],
      w4p, e_b4[None, :], codebooks,
      wd1, d_b1[None, :], w20, w21, w22, w23, d_b2[None, :],
      v0, v1, v2, v3, d_b3[None, :], wd4, d_b4[None, :])

    o0, o1, o2, o3, loss = outs
    xh = jnp.stack([o0, o1, o2, o3], axis=3)
    x_hat = xh.reshape(B, C_IN, 4 * LP)[:, :, :2 * (2 * L + 2) + 2]
    return x_hat, loss[0, 0]
